# baseline (speedup 1.0000x reference)
import numpy as np

import jax
import jax.numpy as jnp
from jax import lax
from jax.experimental import pallas as pl
from jax.experimental.pallas import tpu as pltpu

_BT = 256


def _deltas(n_rows, C, H, row_channel, row_pos):
    d = np.zeros((H, H, C, n_rows), np.float32)
    for r in range(n_rows):
        c, (h, w) = row_channel(r), row_pos(r)
        if c < C:
            d[h, w, c, r] = 1.0
    return jnp.asarray(d, jnp.bfloat16)


_D1 = None
_D2 = None


def _get_deltas():
    global _D1, _D2
    if _D1 is None:
        _D1 = _deltas(3072, 3, 32, lambda r: r // 1024,
                      lambda r: ((r % 1024) // 32, r % 32))
        _D2 = _deltas(1568, 6, 14, lambda r: r % 8,
                      lambda r: (r // 112, (r % 112) // 8))
    return _D1, _D2


def _phase_toeplitz(deltas, w4, Fpad, P):
    Cout, Cin = w4.shape[0], w4.shape[1]
    n = deltas.shape[3]
    w4p = jnp.pad(w4.astype(jnp.bfloat16), ((0, Fpad - Cout), (0, 0), (0, 0), (0, 0)))
    mats = []
    for di in range(2):
        for dj in range(2):
            rhs = jnp.pad(w4p, ((0, 0), (0, 0), (di, 1 - di), (dj, 1 - dj)))
            m = lax.conv_general_dilated(
                deltas, rhs.transpose(2, 3, 1, 0),
                window_strides=(2, 2), padding='VALID',
                dimension_numbers=('HWCN', 'HWIO', 'HWCN'))
            mats.append(m.reshape(P * P * Fpad, n))
    return mats


def _net_kernel(x_ref, aud_ref,
                w1a_ref, w1b_ref, w1c_ref, w1d_ref,
                w2a_ref, w2b_ref, w2c_ref, w2d_ref,
                b1_ref, b2_ref,
                f1w_ref, f1b_ref, f2w_ref, f2b_ref, f3w_ref, f3b_ref,
                a1w_ref, a1b_ref, a2w_ref, a2b_ref, a3w_ref, a3b_ref,
                wfa_ref, wfi_ref, bf_ref, o_ref):
    f32 = jnp.float32
    dn = (((1,), (1,)), ((), ()))

    def tdot(a, w_ref):
        return lax.dot_general(a, w_ref[...], dn, preferred_element_type=f32)

    x = x_ref[...]
    y = jnp.maximum(jnp.maximum(tdot(x, w1a_ref), tdot(x, w1b_ref)),
                    jnp.maximum(tdot(x, w1c_ref), tdot(x, w1d_ref)))
    h1 = jnp.maximum(y + b1_ref[...], 0.0).astype(jnp.bfloat16)

    y = jnp.maximum(jnp.maximum(tdot(h1, w2a_ref), tdot(h1, w2b_ref)),
                    jnp.maximum(tdot(h1, w2c_ref), tdot(h1, w2d_ref)))
    h2 = jnp.maximum(y + b2_ref[...], 0.0)

    t = jnp.maximum(jnp.dot(h2, f1w_ref[...], preferred_element_type=f32)
                    + f1b_ref[...], 0.0)
    t = jnp.maximum(jnp.dot(t, f2w_ref[...], preferred_element_type=f32)
                    + f2b_ref[...], 0.0)
    img = jnp.dot(t, f3w_ref[...], preferred_element_type=f32) + f3b_ref[...]

    a = aud_ref[...]
    a = jnp.maximum(jnp.dot(a, a1w_ref[...], preferred_element_type=f32)
                    + a1b_ref[...], 0.0)
    a = jnp.maximum(jnp.dot(a, a2w_ref[...], preferred_element_type=f32)
                    + a2b_ref[...], 0.0)
    aud = jnp.dot(a, a3w_ref[...], preferred_element_type=f32) + a3b_ref[...]

    o_ref[...] = (jnp.dot(aud, wfa_ref[...], preferred_element_type=f32)
                  + jnp.dot(img, wfi_ref[...], preferred_element_type=f32)
                  + bf_ref[...])


def kernel(image, audio, w1m, b1m, w2m, b2m, fc1_wt, fc1_b2, fc2_wt, fc2_b2,
           fc3_wt, fc3_b2, fc1s_wt, fc1s_b2, fc2s_wt, fc2s_b2, fc3s_wt,
           fc3s_b2, fc_wt, fc_b2):
    B = image.shape[0]
    bt = _BT if B % _BT == 0 else B
    bf16 = jnp.bfloat16

    d1, d2 = _get_deltas()
    x2d = image.astype(bf16).reshape(B, 3 * 32 * 32)
    w1t = _phase_toeplitz(d1, w1m.reshape(6, 3, 5, 5), 8, 14)
    w2t = _phase_toeplitz(d2, w2m.reshape(16, 6, 5, 5), 16, 5)
    b1row = jnp.tile(jnp.pad(b1m.reshape(6), (0, 2)), 196).reshape(1, 1568)
    b2row = jnp.tile(b2m.reshape(16), 25).reshape(1, 400)
    f1p = fc1_wt.reshape(16, 25, 120).transpose(1, 0, 2).reshape(400, 120)
    wfa, wfi = fc_wt[0:9, :], fc_wt[9:18, :]

    operands = [x2d, audio, *w1t, *w2t, b1row, b2row,
                f1p, fc1_b2, fc2_wt, fc2_b2, fc3_wt, fc3_b2,
                fc1s_wt, fc1s_b2, fc2s_wt, fc2s_b2, fc3s_wt, fc3s_b2,
                wfa, wfi, fc_b2]
    in_specs = [pl.BlockSpec((bt, 3072), lambda b: (b, 0)),
                pl.BlockSpec((bt, 10), lambda b: (b, 0))]
    in_specs += [pl.BlockSpec(op.shape, lambda b: (0, 0)) for op in operands[2:]]

    return pl.pallas_call(
        _net_kernel,
        out_shape=jax.ShapeDtypeStruct((B, 9), jnp.float32),
        grid=(B // bt,),
        in_specs=in_specs,
        out_specs=pl.BlockSpec((bt, 9), lambda b: (b, 0)),
        compiler_params=pltpu.CompilerParams(
            dimension_semantics=("parallel",),
            vmem_limit_bytes=56 * 1024 * 1024,
        ),
    )(*operands)


# speedup vs baseline: 1.2606x; 1.2606x over previous
import jax
import jax.numpy as jnp
from jax import lax
from jax.experimental import pallas as pl
from jax.experimental.pallas import tpu as pltpu

_BT = 256


def _shift_expand(base, n, stride):
    R, L = base.shape
    p = jnp.pad(base, ((0, 0), (0, stride)))
    t = jnp.broadcast_to(p[:, None, :], (R, n, L + stride))
    return t.reshape(R, n * (L + stride))[:, :n * L].reshape(R * n, L)


def _phase_toeplitz(w4, Fpad, C, H, P):
    Cout, Cin = w4.shape[0], w4.shape[1]
    w4p = jnp.pad(w4.astype(jnp.bfloat16),
                  ((0, Fpad - Cout), (0, C - Cin), (0, 0), (0, 0)))
    mats = []
    for di in range(2):
        for dj in range(2):
            base = jnp.pad(w4p, ((0, 0), (0, 0), (di, H - 5 - di),
                                 (dj, H - 5 - dj))).reshape(Fpad, C * H * H)
            rows = _shift_expand(base, P, 2 * H)
            rows = _shift_expand(rows, P, 2)
            mats.append(rows)
    return mats


def _net_kernel(x_ref, aud_ref,
                w1a_ref, w1b_ref, w1c_ref, w1d_ref,
                w2a_ref, w2b_ref, w2c_ref, w2d_ref,
                b1_ref, b2_ref,
                f1w_ref, f1b_ref, f2w_ref, f2b_ref, f3w_ref, f3b_ref,
                a1w_ref, a1b_ref, a2w_ref, a2b_ref, a3w_ref, a3b_ref,
                wfa_ref, wfi_ref, bf_ref, o_ref):
    f32 = jnp.float32
    dn = (((1,), (1,)), ((), ()))

    def tdot(a, w_ref):
        return lax.dot_general(a, w_ref[...], dn, preferred_element_type=f32)

    x = x_ref[...]
    y = jnp.maximum(jnp.maximum(tdot(x, w1a_ref), tdot(x, w1b_ref)),
                    jnp.maximum(tdot(x, w1c_ref), tdot(x, w1d_ref)))
    h1 = jnp.maximum(y + b1_ref[...], 0.0).astype(jnp.bfloat16)

    y = jnp.maximum(jnp.maximum(tdot(h1, w2a_ref), tdot(h1, w2b_ref)),
                    jnp.maximum(tdot(h1, w2c_ref), tdot(h1, w2d_ref)))
    h2 = jnp.maximum(y + b2_ref[...], 0.0)

    t = jnp.maximum(jnp.dot(h2, f1w_ref[...], preferred_element_type=f32)
                    + f1b_ref[...], 0.0)
    t = jnp.maximum(jnp.dot(t, f2w_ref[...], preferred_element_type=f32)
                    + f2b_ref[...], 0.0)
    img = jnp.dot(t, f3w_ref[...], preferred_element_type=f32) + f3b_ref[...]

    a = aud_ref[...]
    a = jnp.maximum(jnp.dot(a, a1w_ref[...], preferred_element_type=f32)
                    + a1b_ref[...], 0.0)
    a = jnp.maximum(jnp.dot(a, a2w_ref[...], preferred_element_type=f32)
                    + a2b_ref[...], 0.0)
    aud = jnp.dot(a, a3w_ref[...], preferred_element_type=f32) + a3b_ref[...]

    o_ref[...] = (jnp.dot(aud, wfa_ref[...], preferred_element_type=f32)
                  + jnp.dot(img, wfi_ref[...], preferred_element_type=f32)
                  + bf_ref[...])


def kernel(image, audio, w1m, b1m, w2m, b2m, fc1_wt, fc1_b2, fc2_wt, fc2_b2,
           fc3_wt, fc3_b2, fc1s_wt, fc1s_b2, fc2s_wt, fc2s_b2, fc3s_wt,
           fc3s_b2, fc_wt, fc_b2):
    B = image.shape[0]
    bt = _BT if B % _BT == 0 else B
    bf16 = jnp.bfloat16

    x2d = image.astype(bf16).reshape(B, 3 * 32 * 32)
    w1t = _phase_toeplitz(w1m.reshape(6, 3, 5, 5), 8, 3, 32, 14)
    w2t = _phase_toeplitz(w2m.reshape(16, 6, 5, 5), 16, 8, 14, 5)
    b1row = jnp.repeat(jnp.pad(b1m.reshape(6), (0, 2)), 196).reshape(1, 1568)
    b2row = jnp.repeat(b2m.reshape(16), 25).reshape(1, 400)
    wfa, wfi = fc_wt[0:9, :], fc_wt[9:18, :]

    operands = [x2d, audio, *w1t, *w2t, b1row, b2row,
                fc1_wt, fc1_b2, fc2_wt, fc2_b2, fc3_wt, fc3_b2,
                fc1s_wt, fc1s_b2, fc2s_wt, fc2s_b2, fc3s_wt, fc3s_b2,
                wfa, wfi, fc_b2]
    in_specs = [pl.BlockSpec((bt, 3072), lambda b: (b, 0)),
                pl.BlockSpec((bt, 10), lambda b: (b, 0))]
    in_specs += [pl.BlockSpec(op.shape, lambda b: (0, 0)) for op in operands[2:]]

    return pl.pallas_call(
        _net_kernel,
        out_shape=jax.ShapeDtypeStruct((B, 9), jnp.float32),
        grid=(B // bt,),
        in_specs=in_specs,
        out_specs=pl.BlockSpec((bt, 9), lambda b: (b, 0)),
        compiler_params=pltpu.CompilerParams(
            dimension_semantics=("parallel",),
            vmem_limit_bytes=56 * 1024 * 1024,
        ),
    )(*operands)


# speedup vs baseline: 1.6083x; 1.2758x over previous
import jax
import jax.numpy as jnp
from jax import lax
from jax.experimental import pallas as pl
from jax.experimental.pallas import tpu as pltpu

_BT = 512


def _shift_expand(base, n, stride):
    R, L = base.shape
    p = jnp.pad(base, ((0, 0), (0, stride)))
    t = jnp.broadcast_to(p[:, None, :], (R, n, L + stride))
    return t.reshape(R, n * (L + stride))[:, :n * L].reshape(R * n, L)


def _phase_toeplitz(w4, Fpad, C, H, P):
    Cout, Cin = w4.shape[0], w4.shape[1]
    w4p = jnp.pad(w4.astype(jnp.bfloat16),
                  ((0, Fpad - Cout), (0, C - Cin), (0, 0), (0, 0)))
    mats = []
    for di in range(2):
        for dj in range(2):
            base = jnp.pad(w4p, ((0, 0), (0, 0), (di, H - 5 - di),
                                 (dj, H - 5 - dj))).reshape(Fpad, C * H * H)
            rows = _shift_expand(base, P, 2 * H)
            rows = _shift_expand(rows, P, 2)
            mats.append(rows)
    return mats


def _net_kernel(x_ref, aud_ref,
                w1a_ref, w1b_ref, w1c_ref, w1d_ref,
                w2a_ref, w2b_ref, w2c_ref, w2d_ref,
                b1_ref, b2_ref,
                f1w_ref, f1b_ref, f2w_ref, f2b_ref, f3w_ref, f3b_ref,
                a1w_ref, a1b_ref, a2w_ref, a2b_ref, a3w_ref, a3b_ref,
                wfa_ref, wfi_ref, bf_ref, o_ref):
    f32 = jnp.float32
    dn = (((1,), (1,)), ((), ()))

    def tdot(a, w_ref):
        return lax.dot_general(a, w_ref[...], dn, preferred_element_type=f32)

    x = x_ref[...]
    y = jnp.maximum(jnp.maximum(tdot(x, w1a_ref), tdot(x, w1b_ref)),
                    jnp.maximum(tdot(x, w1c_ref), tdot(x, w1d_ref)))
    h1 = jnp.maximum(y + b1_ref[...], 0.0).astype(jnp.bfloat16)

    y = jnp.maximum(jnp.maximum(tdot(h1, w2a_ref), tdot(h1, w2b_ref)),
                    jnp.maximum(tdot(h1, w2c_ref), tdot(h1, w2d_ref)))
    h2 = jnp.maximum(y + b2_ref[...], 0.0)

    t = jnp.maximum(jnp.dot(h2, f1w_ref[...], preferred_element_type=f32)
                    + f1b_ref[...], 0.0)
    t = jnp.maximum(jnp.dot(t, f2w_ref[...], preferred_element_type=f32)
                    + f2b_ref[...], 0.0)
    img = jnp.dot(t, f3w_ref[...], preferred_element_type=f32) + f3b_ref[...]

    a = aud_ref[...]
    a = jnp.maximum(jnp.dot(a, a1w_ref[...], preferred_element_type=f32)
                    + a1b_ref[...], 0.0)
    a = jnp.maximum(jnp.dot(a, a2w_ref[...], preferred_element_type=f32)
                    + a2b_ref[...], 0.0)
    aud = jnp.dot(a, a3w_ref[...], preferred_element_type=f32) + a3b_ref[...]

    o_ref[...] = (jnp.dot(aud, wfa_ref[...], preferred_element_type=f32)
                  + jnp.dot(img, wfi_ref[...], preferred_element_type=f32)
                  + bf_ref[...])


def kernel(image, audio, w1m, b1m, w2m, b2m, fc1_wt, fc1_b2, fc2_wt, fc2_b2,
           fc3_wt, fc3_b2, fc1s_wt, fc1s_b2, fc2s_wt, fc2s_b2, fc3s_wt,
           fc3s_b2, fc_wt, fc_b2):
    B = image.shape[0]
    bt = _BT if B % _BT == 0 else B
    bf16 = jnp.bfloat16

    x2d = image.astype(bf16).reshape(B, 3 * 32 * 32)
    w1t = _phase_toeplitz(w1m.reshape(6, 3, 5, 5), 6, 3, 32, 14)
    w2t = _phase_toeplitz(w2m.reshape(16, 6, 5, 5), 16, 6, 14, 5)
    b1row = jnp.repeat(b1m.reshape(6), 196).reshape(1, 1176)
    b2row = jnp.repeat(b2m.reshape(16), 25).reshape(1, 400)
    wfa, wfi = fc_wt[0:9, :], fc_wt[9:18, :]

    operands = [x2d, audio, *w1t, *w2t, b1row, b2row,
                fc1_wt, fc1_b2, fc2_wt, fc2_b2, fc3_wt, fc3_b2,
                fc1s_wt, fc1s_b2, fc2s_wt, fc2s_b2, fc3s_wt, fc3s_b2,
                wfa, wfi, fc_b2]
    in_specs = [pl.BlockSpec((bt, 3072), lambda b: (b, 0)),
                pl.BlockSpec((bt, 10), lambda b: (b, 0))]
    in_specs += [pl.BlockSpec(op.shape, lambda b: (0, 0)) for op in operands[2:]]

    return pl.pallas_call(
        _net_kernel,
        out_shape=jax.ShapeDtypeStruct((B, 9), jnp.float32),
        grid=(B // bt,),
        in_specs=in_specs,
        out_specs=pl.BlockSpec((bt, 9), lambda b: (b, 0)),
        compiler_params=pltpu.CompilerParams(
            dimension_semantics=("parallel",),
            vmem_limit_bytes=56 * 1024 * 1024,
        ),
    )(*operands)
